# baseline (speedup 1.0000x reference)
import functools

import jax
import jax.numpy as jnp
from jax import lax
from jax.experimental import pallas as pl
from jax.experimental.pallas import tpu as pltpu
from jax.experimental.pallas import tpu_sc as plsc

_CHUNK = 128
_NW = 32
_NSUB = 16


def _edge_agg(y, src2d, dst2d, zeros, n_pad, chunk, n_seg):
    n, d = y.shape
    n_chunks = src2d.shape[0] // _NW
    n_stage = n_chunks // n_seg
    rows_per_tile = n_pad // _NSUB
    mesh = plsc.VectorSubcoreMesh(core_axis_name="c", subcore_axis_name="s")

    @functools.partial(
        pl.kernel,
        mesh=mesh,
        compiler_params=pltpu.CompilerParams(use_tc_tiling_on_sc=False),
        out_type=jax.ShapeDtypeStruct((2 * n_pad, d), jnp.float32),
        scratch_types=[
            pltpu.VMEM((n_stage, chunk), jnp.int32),
            pltpu.VMEM((n_stage, chunk), jnp.int32),
            pltpu.VMEM((4, chunk, d), jnp.float32),
            pltpu.VMEM_SHARED((n_pad, d), jnp.float32),
            [pltpu.SemaphoreType.DMA] * 4,
            [pltpu.SemaphoreType.DMA] * 4,
        ],
    )
    def k(y_hbm, src_hbm, dst_hbm, z_hbm, out_hbm, src_v, dst_v, rows_v,
          agg_s, gsems, ssems):
        c = lax.axis_index("c")
        s = lax.axis_index("s")
        wid = s * 2 + c
        pltpu.sync_copy(z_hbm, agg_s.at[pl.ds(s * rows_per_tile, rows_per_tile)])
        plsc.subcore_barrier()

        def gather(j, p):
            pltpu.async_copy(y_hbm.at[src_v.at[j]], rows_v.at[p], gsems[p])

        def wait_gather(j, p):
            pltpu.make_async_copy(y_hbm.at[src_v.at[j]], rows_v.at[p],
                                  gsems[p]).wait()

        def scatter(j, p):
            pltpu.async_copy(rows_v.at[p], agg_s.at[dst_v.at[j]], ssems[p],
                             add=True)

        def wait_scatter(j, p):
            pltpu.make_async_copy(rows_v.at[p], agg_s.at[dst_v.at[j]],
                                  ssems[p]).wait()

        for h in range(n_seg):
            base = wid * n_chunks + h * n_stage
            pltpu.sync_copy(src_hbm.at[pl.ds(base, n_stage)], src_v)
            pltpu.sync_copy(dst_hbm.at[pl.ds(base, n_stage)], dst_v)

            gather(0, 0)
            gather(1, 1)
            for j in (0, 1):
                wait_gather(j, j)
                scatter(j, j)
                gather(j + 2, j + 2)

            def body(jj, carry):
                for u in range(4):
                    j = 4 * jj + 2 + u
                    p = (2 + u) % 4
                    wait_gather(j, p)
                    scatter(j, p)
                    wait_scatter(j - 2, (p + 2) % 4)
                    gather(j + 2, (p + 2) % 4)
                return carry

            lax.fori_loop(0, (n_stage - 4) // 4, body, 0)

            for j in (n_stage - 2, n_stage - 1):
                p = j % 4
                wait_gather(j, p)
                scatter(j, p)
                wait_scatter(j - 2, (p + 2) % 4)
            for j in (n_stage - 2, n_stage - 1):
                wait_scatter(j, j % 4)
        plsc.subcore_barrier()
        pltpu.sync_copy(
            agg_s.at[pl.ds(s * rows_per_tile, rows_per_tile)],
            out_hbm.at[pl.ds(c * n_pad + s * rows_per_tile, rows_per_tile)])

    return k(y, src2d, dst2d, zeros)


def _matmul(x, w):
    n, kdim = x.shape
    m = w.shape[1]
    bn = 1000 if n % 1000 == 0 else n

    def body(x_ref, w_ref, o_ref):
        o_ref[...] = jnp.dot(x_ref[...], w_ref[...],
                             preferred_element_type=jnp.float32)

    return pl.pallas_call(
        body,
        grid=(n // bn,),
        in_specs=[
            pl.BlockSpec((bn, kdim), lambda i: (i, 0)),
            pl.BlockSpec((kdim, m), lambda i: (0, 0)),
        ],
        out_specs=pl.BlockSpec((bn, m), lambda i: (i, 0)),
        out_shape=jax.ShapeDtypeStruct((n, m), jnp.float32),
    )(x, w)


def _fuse_mm(y, p0, p1, b, eps, w):
    n, d = y.shape
    m = w.shape[1]
    bn = 1000 if n % 1000 == 0 else n

    def body(y_ref, p0_ref, p1_ref, b_ref, eps_ref, w_ref, o_ref):
        h = ((1.0 + eps_ref[0, 0]) * y_ref[...] + p0_ref[...] + p1_ref[...]
             + b_ref[...])
        h = jnp.maximum(h, 0.0)
        o_ref[...] = jnp.dot(h, w_ref[...], preferred_element_type=jnp.float32)

    return pl.pallas_call(
        body,
        grid=(n // bn,),
        in_specs=[
            pl.BlockSpec((bn, d), lambda i: (i, 0)),
            pl.BlockSpec((bn, d), lambda i: (i, 0)),
            pl.BlockSpec((bn, d), lambda i: (i, 0)),
            pl.BlockSpec((1, d), lambda i: (0, 0)),
            pl.BlockSpec(memory_space=pltpu.SMEM),
            pl.BlockSpec((d, m), lambda i: (0, 0)),
        ],
        out_specs=pl.BlockSpec((bn, m), lambda i: (i, 0)),
        out_shape=jax.ShapeDtypeStruct((n, m), jnp.float32),
    )(y, p0, p1, b.reshape(1, d), eps.reshape(1, 1), w)


def _fuse_logsoftmax(y, p0, p1, b, eps):
    n, d = y.shape
    bn = 1000 if n % 1000 == 0 else n

    def body(y_ref, p0_ref, p1_ref, b_ref, eps_ref, o_ref):
        h = ((1.0 + eps_ref[0, 0]) * y_ref[...] + p0_ref[...] + p1_ref[...]
             + b_ref[...])
        mx = jnp.max(h, axis=1, keepdims=True)
        lse = jnp.log(jnp.sum(jnp.exp(h - mx), axis=1, keepdims=True)) + mx
        o_ref[...] = h - lse

    return pl.pallas_call(
        body,
        grid=(n // bn,),
        in_specs=[
            pl.BlockSpec((bn, d), lambda i: (i, 0)),
            pl.BlockSpec((bn, d), lambda i: (i, 0)),
            pl.BlockSpec((bn, d), lambda i: (i, 0)),
            pl.BlockSpec((1, d), lambda i: (0, 0)),
            pl.BlockSpec(memory_space=pltpu.SMEM),
        ],
        out_specs=pl.BlockSpec((bn, d), lambda i: (i, 0)),
        out_shape=jax.ShapeDtypeStruct((n, d), jnp.float32),
    )(y, p0, p1, b.reshape(1, d), eps.reshape(1, 1))


def kernel(x, edge_index, W1, b1, eps1, W2, b2, eps2):
    n, d = x.shape
    e = edge_index.shape[1]
    h_dim = W1.shape[1]
    c_dim = W2.shape[1]

    n_pad = (n + 1 + _NSUB * 8 - 1) // (_NSUB * 8) * (_NSUB * 8)
    rows_per_tile = n_pad // _NSUB

    epb = _NW * _CHUNK * 80
    e_pad = (e + epb - 1) // epb * epb
    ei = edge_index.astype(jnp.int32)
    src = jnp.concatenate([ei[0], jnp.zeros((e_pad - e,), jnp.int32)])
    dst = jnp.concatenate([ei[1], jnp.full((e_pad - e,), n, jnp.int32)])

    zeros_h = jnp.zeros((rows_per_tile, h_dim), jnp.float32)
    zeros_c = jnp.zeros((rows_per_tile, c_dim), jnp.float32)

    y1 = _matmul(x, W1)
    parts1 = _edge_agg(y1, src.reshape(-1, 64), dst.reshape(-1, 64), zeros_h,
                       n_pad, 64, 2)
    p0 = lax.slice(parts1, (0, 0), (n, h_dim))
    p1 = lax.slice(parts1, (n_pad, 0), (n_pad + n, h_dim))

    y2 = _fuse_mm(y1, p0, p1, b1, eps1, W2)
    parts2 = _edge_agg(y2, src.reshape(-1, _CHUNK), dst.reshape(-1, _CHUNK),
                       zeros_c, n_pad, _CHUNK, 1)
    q0 = lax.slice(parts2, (0, 0), (n, c_dim))
    q1 = lax.slice(parts2, (n_pad, 0), (n_pad + n, c_dim))

    return _fuse_logsoftmax(y2, q0, q1, b2, eps2)


# speedup vs baseline: 1.0290x; 1.0290x over previous
import functools

import jax
import jax.numpy as jnp
from jax import lax
from jax.experimental import pallas as pl
from jax.experimental.pallas import tpu as pltpu
from jax.experimental.pallas import tpu_sc as plsc

_CHUNK = 128
_NW = 32
_NSUB = 16


def _edge_agg(y, src2d, dst2d, zeros, n_pad, chunk, segs0, segs1):
    n, d = y.shape
    t_chunks = src2d.shape[0]
    n_stage = t_chunks // (_NSUB * (segs0 + segs1))
    rows_per_tile = n_pad // _NSUB
    mesh = plsc.VectorSubcoreMesh(core_axis_name="c", subcore_axis_name="s")

    @functools.partial(
        pl.kernel,
        mesh=mesh,
        compiler_params=pltpu.CompilerParams(use_tc_tiling_on_sc=False),
        out_type=jax.ShapeDtypeStruct((2 * n_pad, d), jnp.float32),
        scratch_types=[
            pltpu.VMEM((n_stage, chunk), jnp.int32),
            pltpu.VMEM((n_stage, chunk), jnp.int32),
            pltpu.VMEM((4, chunk, d), jnp.float32),
            pltpu.VMEM_SHARED((n_pad, d), jnp.float32),
            [pltpu.SemaphoreType.DMA] * 4,
            [pltpu.SemaphoreType.DMA] * 4,
        ],
    )
    def k(y_hbm, src_hbm, dst_hbm, z_hbm, out_hbm, src_v, dst_v, rows_v,
          agg_s, gsems, ssems):
        c = lax.axis_index("c")
        s = lax.axis_index("s")
        pltpu.sync_copy(z_hbm, agg_s.at[pl.ds(s * rows_per_tile, rows_per_tile)])
        plsc.subcore_barrier()

        def gather(j, p):
            pltpu.async_copy(y_hbm.at[src_v.at[j]], rows_v.at[p], gsems[p])

        def wait_gather(j, p):
            pltpu.make_async_copy(y_hbm.at[src_v.at[j]], rows_v.at[p],
                                  gsems[p]).wait()

        def scatter(j, p):
            pltpu.async_copy(rows_v.at[p], agg_s.at[dst_v.at[j]], ssems[p],
                             add=True)

        def wait_scatter(j, p):
            pltpu.make_async_copy(rows_v.at[p], agg_s.at[dst_v.at[j]],
                                  ssems[p]).wait()

        def run_segment(base):
            pltpu.sync_copy(src_hbm.at[pl.ds(base, n_stage)], src_v)
            pltpu.sync_copy(dst_hbm.at[pl.ds(base, n_stage)], dst_v)

            gather(0, 0)
            gather(1, 1)
            for j in (0, 1):
                wait_gather(j, j)
                scatter(j, j)
                gather(j + 2, j + 2)

            def body(jj, carry):
                for u in range(4):
                    j = 4 * jj + 2 + u
                    p = (2 + u) % 4
                    wait_gather(j, p)
                    scatter(j, p)
                    wait_scatter(j - 2, (p + 2) % 4)
                    gather(j + 2, (p + 2) % 4)
                return carry

            lax.fori_loop(0, (n_stage - 4) // 4, body, 0)

            for j in (n_stage - 2, n_stage - 1):
                p = j % 4
                wait_gather(j, p)
                scatter(j, p)
                wait_scatter(j - 2, (p + 2) % 4)
            for j in (n_stage - 2, n_stage - 1):
                wait_scatter(j, j % 4)

        @pl.when(c == 0)
        def _():
            for h in range(segs0):
                run_segment((s * segs0 + h) * n_stage)

        @pl.when(c == 1)
        def _():
            for h in range(segs1):
                run_segment((_NSUB * segs0 + s * segs1 + h) * n_stage)

        plsc.subcore_barrier()
        pltpu.sync_copy(
            agg_s.at[pl.ds(s * rows_per_tile, rows_per_tile)],
            out_hbm.at[pl.ds(c * n_pad + s * rows_per_tile, rows_per_tile)])

    return k(y, src2d, dst2d, zeros)


def _matmul(x, w):
    n, kdim = x.shape
    m = w.shape[1]
    bn = 1000 if n % 1000 == 0 else n

    def body(x_ref, w_ref, o_ref):
        o_ref[...] = jnp.dot(x_ref[...], w_ref[...],
                             preferred_element_type=jnp.float32)

    return pl.pallas_call(
        body,
        grid=(n // bn,),
        in_specs=[
            pl.BlockSpec((bn, kdim), lambda i: (i, 0)),
            pl.BlockSpec((kdim, m), lambda i: (0, 0)),
        ],
        out_specs=pl.BlockSpec((bn, m), lambda i: (i, 0)),
        out_shape=jax.ShapeDtypeStruct((n, m), jnp.float32),
    )(x, w)


def _fuse_mm(y, p0, p1, b, eps, w):
    n, d = y.shape
    m = w.shape[1]
    bn = 1000 if n % 1000 == 0 else n

    def body(y_ref, p0_ref, p1_ref, b_ref, eps_ref, w_ref, o_ref):
        h = ((1.0 + eps_ref[0, 0]) * y_ref[...] + p0_ref[...] + p1_ref[...]
             + b_ref[...])
        h = jnp.maximum(h, 0.0)
        o_ref[...] = jnp.dot(h, w_ref[...], preferred_element_type=jnp.float32)

    return pl.pallas_call(
        body,
        grid=(n // bn,),
        in_specs=[
            pl.BlockSpec((bn, d), lambda i: (i, 0)),
            pl.BlockSpec((bn, d), lambda i: (i, 0)),
            pl.BlockSpec((bn, d), lambda i: (i, 0)),
            pl.BlockSpec((1, d), lambda i: (0, 0)),
            pl.BlockSpec(memory_space=pltpu.SMEM),
            pl.BlockSpec((d, m), lambda i: (0, 0)),
        ],
        out_specs=pl.BlockSpec((bn, m), lambda i: (i, 0)),
        out_shape=jax.ShapeDtypeStruct((n, m), jnp.float32),
    )(y, p0, p1, b.reshape(1, d), eps.reshape(1, 1), w)


def _fuse_logsoftmax(y, p0, p1, b, eps):
    n, d = y.shape
    bn = 1000 if n % 1000 == 0 else n

    def body(y_ref, p0_ref, p1_ref, b_ref, eps_ref, o_ref):
        h = ((1.0 + eps_ref[0, 0]) * y_ref[...] + p0_ref[...] + p1_ref[...]
             + b_ref[...])
        mx = jnp.max(h, axis=1, keepdims=True)
        lse = jnp.log(jnp.sum(jnp.exp(h - mx), axis=1, keepdims=True)) + mx
        o_ref[...] = h - lse

    return pl.pallas_call(
        body,
        grid=(n // bn,),
        in_specs=[
            pl.BlockSpec((bn, d), lambda i: (i, 0)),
            pl.BlockSpec((bn, d), lambda i: (i, 0)),
            pl.BlockSpec((bn, d), lambda i: (i, 0)),
            pl.BlockSpec((1, d), lambda i: (0, 0)),
            pl.BlockSpec(memory_space=pltpu.SMEM),
        ],
        out_specs=pl.BlockSpec((bn, d), lambda i: (i, 0)),
        out_shape=jax.ShapeDtypeStruct((n, d), jnp.float32),
    )(y, p0, p1, b.reshape(1, d), eps.reshape(1, 1))


def kernel(x, edge_index, W1, b1, eps1, W2, b2, eps2):
    n, d = x.shape
    e = edge_index.shape[1]
    h_dim = W1.shape[1]
    c_dim = W2.shape[1]

    n_pad = (n + 1 + _NSUB * 8 - 1) // (_NSUB * 8) * (_NSUB * 8)
    rows_per_tile = n_pad // _NSUB

    epb = _NW * _CHUNK * 80
    e_pad = (e + epb - 1) // epb * epb
    ei = edge_index.astype(jnp.int32)
    src = jnp.concatenate([ei[0], jnp.zeros((e_pad - e,), jnp.int32)])
    dst = jnp.concatenate([ei[1], jnp.full((e_pad - e,), n, jnp.int32)])

    zeros_h = jnp.zeros((rows_per_tile, h_dim), jnp.float32)
    zeros_c = jnp.zeros((rows_per_tile, c_dim), jnp.float32)

    y1 = _matmul(x, W1)
    parts1 = _edge_agg(y1, src.reshape(-1, 64), dst.reshape(-1, 64), zeros_h,
                       n_pad, 64, 3, 1)
    p0 = lax.slice(parts1, (0, 0), (n, h_dim))
    p1 = lax.slice(parts1, (n_pad, 0), (n_pad + n, h_dim))

    y2 = _fuse_mm(y1, p0, p1, b1, eps1, W2)
    parts2 = _edge_agg(y2, src.reshape(-1, _CHUNK), dst.reshape(-1, _CHUNK),
                       zeros_c, n_pad, _CHUNK, 3, 1)
    q0 = lax.slice(parts2, (0, 0), (n, c_dim))
    q1 = lax.slice(parts2, (n_pad, 0), (n_pad + n, c_dim))

    return _fuse_logsoftmax(y2, q0, q1, b2, eps2)


# speedup vs baseline: 1.0299x; 1.0009x over previous
import functools

import jax
import jax.numpy as jnp
from jax import lax
from jax.experimental import pallas as pl
from jax.experimental.pallas import tpu as pltpu
from jax.experimental.pallas import tpu_sc as plsc

_CHUNK = 128
_NW = 32
_NSUB = 16


def _edge_agg(y, src2d, dst2d, zeros, n_pad, chunk, segs0, segs1):
    n, d = y.shape
    t_chunks = src2d.shape[0]
    n_stage = t_chunks // (_NSUB * (segs0 + segs1))
    rows_per_tile = n_pad // _NSUB
    mesh = plsc.VectorSubcoreMesh(core_axis_name="c", subcore_axis_name="s")

    @functools.partial(
        pl.kernel,
        mesh=mesh,
        compiler_params=pltpu.CompilerParams(use_tc_tiling_on_sc=False),
        out_type=jax.ShapeDtypeStruct((2 * n_pad, d), jnp.float32),
        scratch_types=[
            pltpu.VMEM((n_stage, chunk), jnp.int32),
            pltpu.VMEM((n_stage, chunk), jnp.int32),
            pltpu.VMEM((4, chunk, d), jnp.float32),
            pltpu.VMEM_SHARED((n_pad, d), jnp.float32),
            [pltpu.SemaphoreType.DMA] * 4,
            [pltpu.SemaphoreType.DMA] * 4,
        ],
    )
    def k(y_hbm, src_hbm, dst_hbm, z_hbm, out_hbm, src_v, dst_v, rows_v,
          agg_s, gsems, ssems):
        c = lax.axis_index("c")
        s = lax.axis_index("s")
        with jax.named_scope("zero"):
            pltpu.sync_copy(z_hbm,
                            agg_s.at[pl.ds(s * rows_per_tile, rows_per_tile)])
            plsc.subcore_barrier()

        def gather(j, p):
            pltpu.async_copy(y_hbm.at[src_v.at[j]], rows_v.at[p], gsems[p])

        def wait_gather(j, p):
            pltpu.make_async_copy(y_hbm.at[src_v.at[j]], rows_v.at[p],
                                  gsems[p]).wait()

        def scatter(j, p):
            pltpu.async_copy(rows_v.at[p], agg_s.at[dst_v.at[j]], ssems[p],
                             add=True)

        def wait_scatter(j, p):
            pltpu.make_async_copy(rows_v.at[p], agg_s.at[dst_v.at[j]],
                                  ssems[p]).wait()

        def run_segment(base):
            with jax.named_scope("stage"):
                pltpu.sync_copy(src_hbm.at[pl.ds(base, n_stage)], src_v)
                pltpu.sync_copy(dst_hbm.at[pl.ds(base, n_stage)], dst_v)

            gather(0, 0)
            gather(1, 1)
            for j in (0, 1):
                wait_gather(j, j)
                scatter(j, j)
                gather(j + 2, j + 2)

            def body(jj, carry):
                for u in range(4):
                    j = 4 * jj + 2 + u
                    p = (2 + u) % 4
                    wait_gather(j, p)
                    scatter(j, p)
                    wait_scatter(j - 2, (p + 2) % 4)
                    gather(j + 2, (p + 2) % 4)
                return carry

            with jax.named_scope("pipe"):
                lax.fori_loop(0, (n_stage - 4) // 4, body, 0)

            for j in (n_stage - 2, n_stage - 1):
                p = j % 4
                wait_gather(j, p)
                scatter(j, p)
                wait_scatter(j - 2, (p + 2) % 4)
            for j in (n_stage - 2, n_stage - 1):
                wait_scatter(j, j % 4)

        @pl.when(c == 0)
        def _():
            for h in range(segs0):
                run_segment((s * segs0 + h) * n_stage)

        @pl.when(c == 1)
        def _():
            for h in range(segs1):
                run_segment((_NSUB * segs0 + s * segs1 + h) * n_stage)

        plsc.subcore_barrier()
        with jax.named_scope("readout"):
            pltpu.sync_copy(
                agg_s.at[pl.ds(s * rows_per_tile, rows_per_tile)],
                out_hbm.at[pl.ds(c * n_pad + s * rows_per_tile, rows_per_tile)])

    return k(y, src2d, dst2d, zeros)


def _matmul(x, w):
    n, kdim = x.shape
    m = w.shape[1]
    bn = 1000 if n % 1000 == 0 else n

    def body(x_ref, w_ref, o_ref):
        o_ref[...] = jnp.dot(x_ref[...], w_ref[...],
                             preferred_element_type=jnp.float32)

    return pl.pallas_call(
        body,
        grid=(n // bn,),
        in_specs=[
            pl.BlockSpec((bn, kdim), lambda i: (i, 0)),
            pl.BlockSpec((kdim, m), lambda i: (0, 0)),
        ],
        out_specs=pl.BlockSpec((bn, m), lambda i: (i, 0)),
        out_shape=jax.ShapeDtypeStruct((n, m), jnp.float32),
    )(x, w)


def _fuse_mm(y, p0, p1, b, eps, w):
    n, d = y.shape
    m = w.shape[1]
    bn = 1000 if n % 1000 == 0 else n

    def body(y_ref, p0_ref, p1_ref, b_ref, eps_ref, w_ref, o_ref):
        h = ((1.0 + eps_ref[0, 0]) * y_ref[...] + p0_ref[...] + p1_ref[...]
             + b_ref[...])
        h = jnp.maximum(h, 0.0)
        o_ref[...] = jnp.dot(h, w_ref[...], preferred_element_type=jnp.float32)

    return pl.pallas_call(
        body,
        grid=(n // bn,),
        in_specs=[
            pl.BlockSpec((bn, d), lambda i: (i, 0)),
            pl.BlockSpec((bn, d), lambda i: (i, 0)),
            pl.BlockSpec((bn, d), lambda i: (i, 0)),
            pl.BlockSpec((1, d), lambda i: (0, 0)),
            pl.BlockSpec(memory_space=pltpu.SMEM),
            pl.BlockSpec((d, m), lambda i: (0, 0)),
        ],
        out_specs=pl.BlockSpec((bn, m), lambda i: (i, 0)),
        out_shape=jax.ShapeDtypeStruct((n, m), jnp.float32),
    )(y, p0, p1, b.reshape(1, d), eps.reshape(1, 1), w)


def _fuse_logsoftmax(y, p0, p1, b, eps):
    n, d = y.shape
    bn = 1000 if n % 1000 == 0 else n

    def body(y_ref, p0_ref, p1_ref, b_ref, eps_ref, o_ref):
        h = ((1.0 + eps_ref[0, 0]) * y_ref[...] + p0_ref[...] + p1_ref[...]
             + b_ref[...])
        mx = jnp.max(h, axis=1, keepdims=True)
        lse = jnp.log(jnp.sum(jnp.exp(h - mx), axis=1, keepdims=True)) + mx
        o_ref[...] = h - lse

    return pl.pallas_call(
        body,
        grid=(n // bn,),
        in_specs=[
            pl.BlockSpec((bn, d), lambda i: (i, 0)),
            pl.BlockSpec((bn, d), lambda i: (i, 0)),
            pl.BlockSpec((bn, d), lambda i: (i, 0)),
            pl.BlockSpec((1, d), lambda i: (0, 0)),
            pl.BlockSpec(memory_space=pltpu.SMEM),
        ],
        out_specs=pl.BlockSpec((bn, d), lambda i: (i, 0)),
        out_shape=jax.ShapeDtypeStruct((n, d), jnp.float32),
    )(y, p0, p1, b.reshape(1, d), eps.reshape(1, 1))


def kernel(x, edge_index, W1, b1, eps1, W2, b2, eps2):
    n, d = x.shape
    e = edge_index.shape[1]
    h_dim = W1.shape[1]
    c_dim = W2.shape[1]

    n_pad = (n + 1 + _NSUB * 8 - 1) // (_NSUB * 8) * (_NSUB * 8)
    rows_per_tile = n_pad // _NSUB

    epb = _NW * _CHUNK * 80
    e_pad = (e + epb - 1) // epb * epb
    ei = edge_index.astype(jnp.int32)
    src = jnp.concatenate([ei[0], jnp.zeros((e_pad - e,), jnp.int32)])
    dst = jnp.concatenate([ei[1], jnp.full((e_pad - e,), n, jnp.int32)])

    zeros_h = jnp.zeros((rows_per_tile, h_dim), jnp.float32)
    zeros_c = jnp.zeros((rows_per_tile, c_dim), jnp.float32)

    y1 = _matmul(x, W1)
    parts1 = _edge_agg(y1, src.reshape(-1, 64), dst.reshape(-1, 64), zeros_h,
                       n_pad, 64, 3, 1)
    p0 = lax.slice(parts1, (0, 0), (n, h_dim))
    p1 = lax.slice(parts1, (n_pad, 0), (n_pad + n, h_dim))

    y2 = _fuse_mm(y1, p0, p1, b1, eps1, W2)
    parts2 = _edge_agg(y2, src.reshape(-1, _CHUNK), dst.reshape(-1, _CHUNK),
                       zeros_c, n_pad, _CHUNK, 3, 1)
    q0 = lax.slice(parts2, (0, 0), (n, c_dim))
    q1 = lax.slice(parts2, (n_pad, 0), (n_pad + n, c_dim))

    return _fuse_logsoftmax(y2, q0, q1, b2, eps2)


# speedup vs baseline: 2.5276x; 2.4542x over previous
import functools

import jax
import jax.numpy as jnp
from jax import lax
from jax.experimental import pallas as pl
from jax.experimental.pallas import tpu as pltpu
from jax.experimental.pallas import tpu_sc as plsc

_CHUNK = 128
_NW = 32
_NSUB = 16


def _edge_agg(y, src2d, dst2d, zeros, n_pad, chunk, segs0, segs1):
    n, d = y.shape
    t_chunks = src2d.shape[0]
    n_stage = t_chunks // (_NSUB * (segs0 + segs1))
    rows_per_tile = n_pad // _NSUB
    mesh = plsc.VectorSubcoreMesh(core_axis_name="c", subcore_axis_name="s")

    @functools.partial(
        pl.kernel,
        mesh=mesh,
        compiler_params=pltpu.CompilerParams(use_tc_tiling_on_sc=False),
        out_type=jax.ShapeDtypeStruct((2 * n_pad, d), jnp.float32),
        scratch_types=[
            pltpu.VMEM((n_stage, chunk), jnp.int32),
            pltpu.VMEM((n_stage, chunk), jnp.int32),
            pltpu.VMEM((4, chunk, d), jnp.float32),
            pltpu.VMEM_SHARED((n_pad, d), jnp.float32),
            [pltpu.SemaphoreType.DMA] * 4,
            [pltpu.SemaphoreType.DMA] * 4,
        ],
    )
    def k(y_hbm, src_hbm, dst_hbm, z_hbm, out_hbm, src_v, dst_v, rows_v,
          agg_s, gsems, ssems):
        c = lax.axis_index("c")
        s = lax.axis_index("s")
        with jax.named_scope("zero"):
            pltpu.sync_copy(z_hbm,
                            agg_s.at[pl.ds(s * rows_per_tile, rows_per_tile)])
            plsc.subcore_barrier()

        def gather(j, p):
            pltpu.async_copy(y_hbm.at[src_v.at[j]], rows_v.at[p], gsems[p])

        def wait_gather(j, p):
            pltpu.make_async_copy(y_hbm.at[src_v.at[j]], rows_v.at[p],
                                  gsems[p]).wait()

        def scatter(j, p):
            pltpu.async_copy(rows_v.at[p], agg_s.at[dst_v.at[j]], ssems[p],
                             add=True)

        def wait_scatter(j, p):
            pltpu.make_async_copy(rows_v.at[p], agg_s.at[dst_v.at[j]],
                                  ssems[p]).wait()

        def run_segment(base):
            with jax.named_scope("stage"):
                pltpu.sync_copy(src_hbm.at[pl.ds(base, n_stage)], src_v)
                pltpu.sync_copy(dst_hbm.at[pl.ds(base, n_stage)], dst_v)

            gather(0, 0)
            gather(1, 1)
            for j in (0, 1):
                wait_gather(j, j)
                scatter(j, j)
                gather(j + 2, j + 2)

            def body(jj, carry):
                for u in range(4):
                    j = 4 * jj + 2 + u
                    p = (2 + u) % 4
                    wait_gather(j, p)
                    scatter(j, p)
                    wait_scatter(j - 2, (p + 2) % 4)
                    gather(j + 2, (p + 2) % 4)
                return carry

            with jax.named_scope("pipe"):
                lax.fori_loop(0, (n_stage - 4) // 4, body, 0)

            for j in (n_stage - 2, n_stage - 1):
                p = j % 4
                wait_gather(j, p)
                scatter(j, p)
                wait_scatter(j - 2, (p + 2) % 4)
            for j in (n_stage - 2, n_stage - 1):
                wait_scatter(j, j % 4)

        @pl.when(c == 0)
        def _():
            for h in range(segs0):
                run_segment((s * segs0 + h) * n_stage)

        @pl.when(c == 1)
        def _():
            for h in range(segs1):
                run_segment((_NSUB * segs0 + s * segs1 + h) * n_stage)

        plsc.subcore_barrier()
        with jax.named_scope("readout"):
            pltpu.sync_copy(
                agg_s.at[pl.ds(s * rows_per_tile, rows_per_tile)],
                out_hbm.at[pl.ds(c * n_pad + s * rows_per_tile, rows_per_tile)])

    return k(y, src2d, dst2d, zeros)


def _matmul(x, w):
    n, kdim = x.shape
    m = w.shape[1]
    bn = 1000 if n % 1000 == 0 else n

    def body(x_ref, w_ref, o_ref):
        o_ref[...] = jnp.dot(x_ref[...], w_ref[...],
                             preferred_element_type=jnp.float32)

    return pl.pallas_call(
        body,
        grid=(n // bn,),
        in_specs=[
            pl.BlockSpec((bn, kdim), lambda i: (i, 0)),
            pl.BlockSpec((kdim, m), lambda i: (0, 0)),
        ],
        out_specs=pl.BlockSpec((bn, m), lambda i: (i, 0)),
        out_shape=jax.ShapeDtypeStruct((n, m), jnp.float32),
    )(x, w)


def _fuse_mm(y, p0, p1, b, eps, w):
    n, d = y.shape
    m = w.shape[1]
    bn = 1000 if n % 1000 == 0 else n

    def body(y_ref, p0_ref, p1_ref, b_ref, eps_ref, w_ref, o_ref):
        h = ((1.0 + eps_ref[0, 0]) * y_ref[...] + p0_ref[...] + p1_ref[...]
             + b_ref[...])
        h = jnp.maximum(h, 0.0)
        o_ref[...] = jnp.dot(h, w_ref[...], preferred_element_type=jnp.float32)

    return pl.pallas_call(
        body,
        grid=(n // bn,),
        in_specs=[
            pl.BlockSpec((bn, d), lambda i: (i, 0)),
            pl.BlockSpec((bn, d), lambda i: (i, 0)),
            pl.BlockSpec((bn, d), lambda i: (i, 0)),
            pl.BlockSpec((1, d), lambda i: (0, 0)),
            pl.BlockSpec(memory_space=pltpu.SMEM),
            pl.BlockSpec((d, m), lambda i: (0, 0)),
        ],
        out_specs=pl.BlockSpec((bn, m), lambda i: (i, 0)),
        out_shape=jax.ShapeDtypeStruct((n, m), jnp.float32),
    )(y, p0, p1, b.reshape(1, d), eps.reshape(1, 1), w)


def _fuse_logsoftmax(y, p0, p1, b, eps):
    n, d = y.shape
    bn = 1000 if n % 1000 == 0 else n

    def body(y_ref, p0_ref, p1_ref, b_ref, eps_ref, o_ref):
        h = ((1.0 + eps_ref[0, 0]) * y_ref[...] + p0_ref[...] + p1_ref[...]
             + b_ref[...])
        mx = jnp.max(h, axis=1, keepdims=True)
        lse = jnp.log(jnp.sum(jnp.exp(h - mx), axis=1, keepdims=True)) + mx
        o_ref[...] = h - lse

    return pl.pallas_call(
        body,
        grid=(n // bn,),
        in_specs=[
            pl.BlockSpec((bn, d), lambda i: (i, 0)),
            pl.BlockSpec((bn, d), lambda i: (i, 0)),
            pl.BlockSpec((bn, d), lambda i: (i, 0)),
            pl.BlockSpec((1, d), lambda i: (0, 0)),
            pl.BlockSpec(memory_space=pltpu.SMEM),
        ],
        out_specs=pl.BlockSpec((bn, d), lambda i: (i, 0)),
        out_shape=jax.ShapeDtypeStruct((n, d), jnp.float32),
    )(y, p0, p1, b.reshape(1, d), eps.reshape(1, 1))


def kernel(x, edge_index, W1, b1, eps1, W2, b2, eps2):
    n, d = x.shape
    e = edge_index.shape[1]
    h_dim = W1.shape[1]
    c_dim = W2.shape[1]

    n_pad = (n + 1 + _NSUB * 8 - 1) // (_NSUB * 8) * (_NSUB * 8)
    rows_per_tile = n_pad // _NSUB

    epb = _NW * _CHUNK * 80
    e_pad = (e + epb - 1) // epb * epb
    ei = edge_index.astype(jnp.int32)
    pad_i = jnp.arange(e_pad - e, dtype=jnp.int32)
    src = jnp.concatenate([ei[0], pad_i % n])
    dst = jnp.concatenate([ei[1], n + pad_i % (n_pad - n)])

    zeros_h = jnp.zeros((rows_per_tile, h_dim), jnp.float32)
    zeros_c = jnp.zeros((rows_per_tile, c_dim), jnp.float32)

    y1 = _matmul(x, W1)
    parts1 = _edge_agg(y1, src.reshape(-1, 64), dst.reshape(-1, 64), zeros_h,
                       n_pad, 64, 2, 2)
    p0 = lax.slice(parts1, (0, 0), (n, h_dim))
    p1 = lax.slice(parts1, (n_pad, 0), (n_pad + n, h_dim))

    y2 = _fuse_mm(y1, p0, p1, b1, eps1, W2)
    parts2 = _edge_agg(y2, src.reshape(-1, _CHUNK), dst.reshape(-1, _CHUNK),
                       zeros_c, n_pad, _CHUNK, 2, 2)
    q0 = lax.slice(parts2, (0, 0), (n, c_dim))
    q1 = lax.slice(parts2, (n_pad, 0), (n_pad + n, c_dim))

    return _fuse_logsoftmax(y2, q0, q1, b2, eps2)


# speedup vs baseline: 2.6588x; 1.0519x over previous
import functools

import jax
import jax.numpy as jnp
from jax import lax
from jax.experimental import pallas as pl
from jax.experimental.pallas import tpu as pltpu
from jax.experimental.pallas import tpu_sc as plsc

_CHUNK = 128
_NW = 32
_NSUB = 16


def _edge_agg(y, src2d, dst2d, zeros, n_pad, chunk, segs0, segs1):
    n, d = y.shape
    t_chunks = src2d.shape[0]
    n_stage = t_chunks // (_NSUB * (segs0 + segs1))
    rows_per_tile = n_pad // _NSUB
    mesh = plsc.VectorSubcoreMesh(core_axis_name="c", subcore_axis_name="s")

    @functools.partial(
        pl.kernel,
        mesh=mesh,
        compiler_params=pltpu.CompilerParams(use_tc_tiling_on_sc=False),
        out_type=jax.ShapeDtypeStruct((2 * n_pad, d), jnp.float32),
        scratch_types=[
            pltpu.VMEM((n_stage, chunk), jnp.int32),
            pltpu.VMEM((n_stage, chunk), jnp.int32),
            pltpu.VMEM((4, chunk, d), jnp.float32),
            pltpu.VMEM_SHARED((n_pad, d), jnp.float32),
            [pltpu.SemaphoreType.DMA] * 4,
            [pltpu.SemaphoreType.DMA] * 4,
        ],
    )
    def k(y_hbm, src_hbm, dst_hbm, z_hbm, out_hbm, src_v, dst_v, rows_v,
          agg_s, gsems, ssems):
        c = lax.axis_index("c")
        s = lax.axis_index("s")
        with jax.named_scope("zero"):
            pltpu.sync_copy(z_hbm,
                            agg_s.at[pl.ds(s * rows_per_tile, rows_per_tile)])
            plsc.subcore_barrier()

        def gather(j, p):
            pltpu.async_copy(y_hbm.at[src_v.at[j]], rows_v.at[p], gsems[p])

        def wait_gather(j, p):
            pltpu.make_async_copy(y_hbm.at[src_v.at[j]], rows_v.at[p],
                                  gsems[p]).wait()

        def scatter(j, p):
            pltpu.async_copy(rows_v.at[p], agg_s.at[dst_v.at[j]], ssems[p],
                             add=True)

        def wait_scatter(j, p):
            pltpu.make_async_copy(rows_v.at[p], agg_s.at[dst_v.at[j]],
                                  ssems[p]).wait()

        def run_segment(base):
            with jax.named_scope("stage"):
                pltpu.sync_copy(src_hbm.at[pl.ds(base, n_stage)], src_v)
                pltpu.sync_copy(dst_hbm.at[pl.ds(base, n_stage)], dst_v)

            gather(0, 0)
            gather(1, 1)
            for j in (0, 1):
                wait_gather(j, j)
                scatter(j, j)
                gather(j + 2, j + 2)

            def body(jj, carry):
                for u in range(4):
                    j = 4 * jj + 2 + u
                    p = (2 + u) % 4
                    wait_gather(j, p)
                    scatter(j, p)
                    wait_scatter(j - 2, (p + 2) % 4)
                    gather(j + 2, (p + 2) % 4)
                return carry

            with jax.named_scope("pipe"):
                lax.fori_loop(0, (n_stage - 4) // 4, body, 0)

            for j in (n_stage - 2, n_stage - 1):
                p = j % 4
                wait_gather(j, p)
                scatter(j, p)
                wait_scatter(j - 2, (p + 2) % 4)
            for j in (n_stage - 2, n_stage - 1):
                wait_scatter(j, j % 4)

        @pl.when(c == 0)
        def _():
            for h in range(segs0):
                run_segment((s * segs0 + h) * n_stage)

        @pl.when(c == 1)
        def _():
            for h in range(segs1):
                run_segment((_NSUB * segs0 + s * segs1 + h) * n_stage)

        plsc.subcore_barrier()
        with jax.named_scope("readout"):
            pltpu.sync_copy(
                agg_s.at[pl.ds(s * rows_per_tile, rows_per_tile)],
                out_hbm.at[pl.ds(c * n_pad + s * rows_per_tile, rows_per_tile)])

    return k(y, src2d, dst2d, zeros)


def _matmul(x, w):
    n, kdim = x.shape
    m = w.shape[1]
    bn = 1000 if n % 1000 == 0 else n

    def body(x_ref, w_ref, o_ref):
        o_ref[...] = jnp.dot(x_ref[...], w_ref[...],
                             preferred_element_type=jnp.float32)

    return pl.pallas_call(
        body,
        grid=(n // bn,),
        in_specs=[
            pl.BlockSpec((bn, kdim), lambda i: (i, 0)),
            pl.BlockSpec((kdim, m), lambda i: (0, 0)),
        ],
        out_specs=pl.BlockSpec((bn, m), lambda i: (i, 0)),
        out_shape=jax.ShapeDtypeStruct((n, m), jnp.float32),
    )(x, w)


def _fuse_mm(y, parts, b, eps, w, n_pad):
    n, d = y.shape
    m = w.shape[1]
    bn = 1024
    nb = n_pad // bn

    def body(y_ref, p0_ref, p1_ref, b_ref, eps_ref, w_ref, o_ref):
        h = ((1.0 + eps_ref[0, 0]) * y_ref[...] + p0_ref[...] + p1_ref[...]
             + b_ref[...])
        h = jnp.maximum(h, 0.0)
        o_ref[...] = jnp.dot(h, w_ref[...], preferred_element_type=jnp.float32)

    return pl.pallas_call(
        body,
        grid=(pl.cdiv(n, bn),),
        in_specs=[
            pl.BlockSpec((bn, d), lambda i: (i, 0)),
            pl.BlockSpec((bn, d), lambda i: (i, 0)),
            pl.BlockSpec((bn, d), lambda i: (nb + i, 0)),
            pl.BlockSpec((1, d), lambda i: (0, 0)),
            pl.BlockSpec(memory_space=pltpu.SMEM),
            pl.BlockSpec((d, m), lambda i: (0, 0)),
        ],
        out_specs=pl.BlockSpec((bn, m), lambda i: (i, 0)),
        out_shape=jax.ShapeDtypeStruct((n, m), jnp.float32),
    )(y, parts, parts, b.reshape(1, d), eps.reshape(1, 1), w)


def _fuse_logsoftmax(y, parts, b, eps, n_pad):
    n, d = y.shape
    bn = 1024
    nb = n_pad // bn

    def body(y_ref, p0_ref, p1_ref, b_ref, eps_ref, o_ref):
        h = ((1.0 + eps_ref[0, 0]) * y_ref[...] + p0_ref[...] + p1_ref[...]
             + b_ref[...])
        mx = jnp.max(h, axis=1, keepdims=True)
        lse = jnp.log(jnp.sum(jnp.exp(h - mx), axis=1, keepdims=True)) + mx
        o_ref[...] = h - lse

    return pl.pallas_call(
        body,
        grid=(pl.cdiv(n, bn),),
        in_specs=[
            pl.BlockSpec((bn, d), lambda i: (i, 0)),
            pl.BlockSpec((bn, d), lambda i: (i, 0)),
            pl.BlockSpec((bn, d), lambda i: (nb + i, 0)),
            pl.BlockSpec((1, d), lambda i: (0, 0)),
            pl.BlockSpec(memory_space=pltpu.SMEM),
        ],
        out_specs=pl.BlockSpec((bn, d), lambda i: (i, 0)),
        out_shape=jax.ShapeDtypeStruct((n, d), jnp.float32),
    )(y, parts, parts, b.reshape(1, d), eps.reshape(1, 1))


def kernel(x, edge_index, W1, b1, eps1, W2, b2, eps2):
    n, d = x.shape
    e = edge_index.shape[1]
    h_dim = W1.shape[1]
    c_dim = W2.shape[1]

    n_pad = (n + 1 + 1024 - 1) // 1024 * 1024
    rows_per_tile = n_pad // _NSUB

    epb = _NW * _CHUNK * 80
    e_pad = (e + epb - 1) // epb * epb
    ei = edge_index.astype(jnp.int32)
    pad_i = jnp.arange(e_pad - e, dtype=jnp.int32)
    src = jnp.concatenate([ei[0], pad_i % n])
    dst = jnp.concatenate([ei[1], n + pad_i % (n_pad - n)])

    zeros_h = jnp.zeros((rows_per_tile, h_dim), jnp.float32)
    zeros_c = jnp.zeros((rows_per_tile, c_dim), jnp.float32)

    y1 = _matmul(x, W1)
    parts1 = _edge_agg(y1, src.reshape(-1, 64), dst.reshape(-1, 64), zeros_h,
                       n_pad, 64, 2, 2)

    y2 = _fuse_mm(y1, parts1, b1, eps1, W2, n_pad)
    parts2 = _edge_agg(y2, src.reshape(-1, _CHUNK), dst.reshape(-1, _CHUNK),
                       zeros_c, n_pad, _CHUNK, 2, 2)

    return _fuse_logsoftmax(y2, parts2, b2, eps2, n_pad)
